# baseline (speedup 1.0000x reference)
import dataclasses
import functools

import jax
import jax.numpy as jnp
from jax import lax
from jax.experimental import pallas as pl
from jax.experimental.pallas import tpu as pltpu
from jax.experimental.pallas import tpu_sc as plsc

N = 10000
E = 320000
D = 128
NC, NS = 2, 16
NW = NC * NS
EPW = E // NW
WIN = 128
NWIN = 80
NHALF = NWIN // 2
EPW_PAD = NWIN * WIN
N_PAD = 10112
ZROWS = N_PAD // NS

@functools.lru_cache(maxsize=None)
def _get_mesh():
    return plsc.VectorSubcoreMesh(core_axis_name="c", subcore_axis_name="s")

_sc_params = pltpu.CompilerParams()
if "needs_layout_passes" in pltpu.CompilerParams.__dataclass_fields__:
    _sc_params = dataclasses.replace(_sc_params, needs_layout_passes=False)


def _sc_degree(ei2):
    @functools.partial(
        pl.kernel,
        out_type=jax.ShapeDtypeStruct((NW, N), jnp.float32),
        mesh=_get_mesh(),
        compiler_params=_sc_params,
        scratch_types=[
            pltpu.VMEM((EPW,), jnp.int32),
            pltpu.VMEM((N,), jnp.float32),
        ],
    )
    def k(ei_hbm, out_hbm, colv, degv):
        c = lax.axis_index("c")
        s = lax.axis_index("s")
        wid = s * NC + c
        pltpu.sync_copy(ei_hbm.at[1, wid], colv)

        @pl.loop(0, N, step=16)
        def _zero(i):
            degv[pl.ds(i, 16)] = jnp.zeros((16,), jnp.float32)

        ones = jnp.full((16,), 1.0, jnp.float32)

        @pl.loop(0, EPW, step=16)
        def _hist(i):
            plsc.addupdate_scatter(degv, [colv[pl.ds(i, 16)]], ones)

        pltpu.sync_copy(degv, out_hbm.at[wid])

    return k(ei2)


def _sc_aggregate(hs, ei4, zeros):
    @functools.partial(
        pl.kernel,
        out_type=jax.ShapeDtypeStruct((NC, N_PAD, D), jnp.float32),
        mesh=_get_mesh(),
        scratch_types=[
            pltpu.VMEM_SHARED((N_PAD, D), jnp.float32),
            pltpu.VMEM((NHALF, WIN), jnp.int32),
            pltpu.VMEM((NHALF, WIN), jnp.int32),
            pltpu.VMEM((WIN, D), jnp.float32),
            pltpu.VMEM((WIN, D), jnp.float32),
            pltpu.SemaphoreType.DMA,
            pltpu.SemaphoreType.DMA,
        ],
    )
    def k(hs_hbm, ei_hbm, zero_hbm, out_hbm, acc, rowi, coli,
          rows0, rows1, gsem, zsem):
        c = lax.axis_index("c")
        s = lax.axis_index("s")
        wid = s * NC + c
        zcp = pltpu.async_copy(zero_hbm,
                               acc.at[pl.ds(s * ZROWS, ZROWS)], zsem)
        pltpu.sync_copy(ei_hbm.at[0, wid, pl.ds(0, NHALF)], rowi)
        pltpu.sync_copy(ei_hbm.at[1, wid, pl.ds(0, NHALF)], coli)
        pltpu.async_copy(hs_hbm.at[rowi.at[0]], rows0, gsem)
        zcp.wait()
        plsc.subcore_barrier()

        for h in range(2):
            if h == 1:
                pltpu.sync_copy(ei_hbm.at[0, wid, pl.ds(NHALF, NHALF)], rowi)
                pltpu.sync_copy(ei_hbm.at[1, wid, pl.ds(NHALF, NHALF)], coli)
                pltpu.async_copy(hs_hbm.at[rowi.at[0]], rows0, gsem)

            @pl.loop(0, NHALF, step=2)
            def _win(j):
                pltpu.make_async_copy(hs_hbm.at[rowi.at[j]], rows0, gsem).wait()
                pltpu.async_copy(hs_hbm.at[rowi.at[j + 1]], rows1, gsem)
                pltpu.sync_copy(rows0, acc.at[coli.at[j]], add=True)
                pltpu.make_async_copy(hs_hbm.at[rowi.at[j + 1]], rows1,
                                      gsem).wait()

                @pl.when(j + 2 < NHALF)
                def _prefetch():
                    pltpu.async_copy(hs_hbm.at[rowi.at[j + 2]], rows0, gsem)

                pltpu.sync_copy(rows1, acc.at[coli.at[j + 1]], add=True)

        plsc.subcore_barrier()
        pltpu.sync_copy(acc.at[pl.ds(s * ZROWS, ZROWS)],
                        out_hbm.at[c, pl.ds(s * ZROWS, ZROWS)])

    return k(hs, ei4, zeros)


def _dis_from(degt_ref):
    deg = jnp.sum(degt_ref[...], axis=1, keepdims=True) + 1.0
    return lax.rsqrt(deg)


def _matmul_t(a, w):
    return lax.dot_general(a, w, (((1,), (1,)), ((), ())),
                           precision=lax.Precision.HIGHEST)


def _bn_lrelu(t, g_ref, be_ref):
    mean = jnp.mean(t, axis=0, keepdims=True)
    var = jnp.mean((t - mean) ** 2, axis=0, keepdims=True)
    t = (t - mean) * lax.rsqrt(var + 1e-5) * g_ref[...] + be_ref[...]
    return jnp.where(t >= 0, t, 0.1 * t)


def _tc_k1a(x, w1, b1):
    BN, NB = 1000, 10

    def body(x_ref, w_ref, b_ref, o_ref):
        o_ref[...] = _matmul_t(x_ref[...], w_ref[...]) + b_ref[...]

    return pl.pallas_call(
        body,
        grid=(NB,),
        in_specs=[
            pl.BlockSpec((BN, D), lambda i: (i, 0)),
            pl.BlockSpec((D, D), lambda i: (0, 0)),
            pl.BlockSpec((1, D), lambda i: (0, 0)),
        ],
        out_specs=pl.BlockSpec((BN, D), lambda i: (i, 0)),
        out_shape=jax.ShapeDtypeStruct((N, D), jnp.float32),
    )(x, w1, b1)


def _tc_k1b(h1raw, degt):
    def body(h_ref, degt_ref, o_ref):
        o_ref[...] = h_ref[...] * _dis_from(degt_ref)

    return pl.pallas_call(
        body, out_shape=jax.ShapeDtypeStruct((N, D), jnp.float32),
    )(h1raw, degt)


_BN, _NB = 1000, 10


def _combine_phase1(i, p_ref, hs_ref, dis, t_ref, stat_ref):
    t = (p_ref[0][0] + p_ref[1][0] + hs_ref[...]) * dis
    t_ref[pl.ds(i * _BN, _BN), :] = t

    @pl.when(i == 0)
    def _init():
        stat_ref[...] = jnp.zeros((8, D), jnp.float32)

    stat_ref[0:1, :] += jnp.sum(t, axis=0, keepdims=True)
    stat_ref[1:2, :] += jnp.sum(t * t, axis=0, keepdims=True)


def _norm_phase2(j, g_ref, be_ref, t_ref, stat_ref):
    mean = stat_ref[0:1, :] * (1.0 / N)
    var = stat_ref[1:2, :] * (1.0 / N) - mean * mean
    t = t_ref[pl.ds(j * _BN, _BN), :]
    t = (t - mean) * lax.rsqrt(var + 1e-5) * g_ref[...] + be_ref[...]
    return jnp.where(t >= 0, t, 0.1 * t)


def _bn_specs():
    return [
        pl.BlockSpec((1, _BN, D), lambda i: (0, jnp.minimum(i, _NB - 1), 0)),
        pl.BlockSpec((1, _BN, D), lambda i: (1, jnp.minimum(i, _NB - 1), 0)),
        pl.BlockSpec((_BN, D), lambda i: (jnp.minimum(i, _NB - 1), 0)),
        pl.BlockSpec((_BN, NW), lambda i: (jnp.where(i < _NB, i, i - _NB), 0)),
        pl.BlockSpec((1, D), lambda i: (0, 0)),
        pl.BlockSpec((1, D), lambda i: (0, 0)),
    ]


def _tc_k2(parts, hs1, degt, g1, be1, w2, b2):
    def body(p0_ref, p1_ref, hs_ref, degt_ref, g_ref, be_ref, w_ref, b_ref,
             o_ref, t_ref, stat_ref):
        i = pl.program_id(0)
        dis = _dis_from(degt_ref)

        @pl.when(i < _NB)
        def _phase1():
            _combine_phase1(i, (p0_ref, p1_ref), hs_ref, dis, t_ref, stat_ref)

        @pl.when(i >= _NB)
        def _phase2():
            t = _norm_phase2(i - _NB, g_ref, be_ref, t_ref, stat_ref)
            o_ref[...] = (_matmul_t(t, w_ref[...]) + b_ref[...]) * dis

    return pl.pallas_call(
        body,
        grid=(2 * _NB,),
        in_specs=_bn_specs() + [
            pl.BlockSpec((D, D), lambda i: (0, 0)),
            pl.BlockSpec((1, D), lambda i: (0, 0)),
        ],
        out_specs=pl.BlockSpec((_BN, D), lambda i: (jnp.maximum(i - _NB, 0), 0)),
        out_shape=jax.ShapeDtypeStruct((N, D), jnp.float32),
        scratch_shapes=[
            pltpu.VMEM((N, D), jnp.float32),
            pltpu.VMEM((8, D), jnp.float32),
        ],
    )(parts, parts, hs1, degt, g1, be1, w2, b2)


def _tc_k3(parts, hs2, degt, g2, be2):
    def body(p0_ref, p1_ref, hs_ref, degt_ref, g_ref, be_ref,
             o_ref, t_ref, stat_ref):
        i = pl.program_id(0)

        @pl.when(i < _NB)
        def _phase1():
            dis = _dis_from(degt_ref)
            _combine_phase1(i, (p0_ref, p1_ref), hs_ref, dis, t_ref, stat_ref)

        @pl.when(i >= _NB)
        def _phase2():
            o_ref[...] = _norm_phase2(i - _NB, g_ref, be_ref, t_ref, stat_ref)

    return pl.pallas_call(
        body,
        grid=(2 * _NB,),
        in_specs=_bn_specs(),
        out_specs=pl.BlockSpec((_BN, D), lambda i: (jnp.maximum(i - _NB, 0), 0)),
        out_shape=jax.ShapeDtypeStruct((N, D), jnp.float32),
        scratch_shapes=[
            pltpu.VMEM((N, D), jnp.float32),
            pltpu.VMEM((8, D), jnp.float32),
        ],
    )(parts, parts, hs2, degt, g2, be2)


def kernel(x, edge_index, W1, b1, g1, be1, W2, b2, g2, be2):
    ei2 = edge_index.reshape(2, NW, EPW)

    npad = EPW_PAD - EPW
    ar = jnp.arange(npad, dtype=jnp.int32)
    pads = jnp.broadcast_to(
        jnp.stack([(ar * 89) % N, N + ar % (N_PAD - N)])[:, None, :],
        (2, NW, npad))
    ei4 = jnp.concatenate([ei2, pads], axis=2).reshape(2, NW, NWIN, WIN)

    zeros = jnp.zeros((ZROWS, D), jnp.float32)
    b1r, g1r, be1r = b1.reshape(1, D), g1.reshape(1, D), be1.reshape(1, D)
    b2r, g2r, be2r = b2.reshape(1, D), g2.reshape(1, D), be2.reshape(1, D)

    h1raw = _tc_k1a(x, W1, b1r)
    degp = _sc_degree(ei2)
    degt = degp.T

    hs1 = _tc_k1b(h1raw, degt)
    p1 = _sc_aggregate(hs1, ei4, zeros)
    hs2 = _tc_k2(p1, hs1, degt, g1r, be1r, W2, b2r)
    p2 = _sc_aggregate(hs2, ei4, zeros)
    return _tc_k3(p2, hs2, degt, g2r, be2r)


# speedup vs baseline: 1.0534x; 1.0534x over previous
import dataclasses
import functools

import jax
import jax.numpy as jnp
from jax import lax
from jax.experimental import pallas as pl
from jax.experimental.pallas import tpu as pltpu
from jax.experimental.pallas import tpu_sc as plsc

N = 10000
E = 320000
D = 128
NC, NS = 2, 16
NW = NC * NS
EPW = E // NW
WIN = 128
NWIN = 80
NHALF = NWIN // 2
EPW_PAD = NWIN * WIN
N_PAD = 10112
ZROWS = N_PAD // NS

@functools.lru_cache(maxsize=None)
def _get_mesh():
    return plsc.VectorSubcoreMesh(core_axis_name="c", subcore_axis_name="s")

_sc_params = pltpu.CompilerParams()
if "needs_layout_passes" in pltpu.CompilerParams.__dataclass_fields__:
    _sc_params = dataclasses.replace(_sc_params, needs_layout_passes=False)


def _sc_degree(ei2):
    @functools.partial(
        pl.kernel,
        out_type=jax.ShapeDtypeStruct((NW, N), jnp.float32),
        mesh=_get_mesh(),
        compiler_params=_sc_params,
        scratch_types=[
            pltpu.VMEM((EPW,), jnp.int32),
            pltpu.VMEM((N,), jnp.float32),
        ],
    )
    def k(ei_hbm, out_hbm, colv, degv):
        c = lax.axis_index("c")
        s = lax.axis_index("s")
        wid = s * NC + c
        pltpu.sync_copy(ei_hbm.at[1, wid], colv)

        @pl.loop(0, N, step=16)
        def _zero(i):
            degv[pl.ds(i, 16)] = jnp.zeros((16,), jnp.float32)

        ones = jnp.full((16,), 1.0, jnp.float32)

        @pl.loop(0, EPW, step=16)
        def _hist(i):
            plsc.addupdate_scatter(degv, [colv[pl.ds(i, 16)]], ones)

        pltpu.sync_copy(degv, out_hbm.at[wid])

    return k(ei2)


def _sc_aggregate(hs, ei4, zeros):
    @functools.partial(
        pl.kernel,
        out_type=jax.ShapeDtypeStruct((NC, N_PAD, D), jnp.float32),
        mesh=_get_mesh(),
        scratch_types=[
            pltpu.VMEM_SHARED((N_PAD, D), jnp.float32),
            pltpu.VMEM((NHALF, WIN), jnp.int32),
            pltpu.VMEM((NHALF, WIN), jnp.int32),
            pltpu.VMEM((WIN, D), jnp.float32),
            pltpu.VMEM((WIN, D), jnp.float32),
            pltpu.SemaphoreType.DMA,
            pltpu.SemaphoreType.DMA,
        ],
    )
    def k(hs_hbm, ei_hbm, zero_hbm, out_hbm, acc, rowi, coli,
          rows0, rows1, gsem, zsem):
        c = lax.axis_index("c")
        s = lax.axis_index("s")
        wid = s * NC + c
        zcp = pltpu.async_copy(zero_hbm,
                               acc.at[pl.ds(s * ZROWS, ZROWS)], zsem)
        pltpu.sync_copy(ei_hbm.at[0, wid, pl.ds(0, NHALF)], rowi)
        pltpu.sync_copy(ei_hbm.at[1, wid, pl.ds(0, NHALF)], coli)
        pltpu.async_copy(hs_hbm.at[rowi.at[0]], rows0, gsem)
        zcp.wait()
        plsc.subcore_barrier()

        for h in range(2):
            if h == 1:
                pltpu.sync_copy(ei_hbm.at[0, wid, pl.ds(NHALF, NHALF)], rowi)
                pltpu.sync_copy(ei_hbm.at[1, wid, pl.ds(NHALF, NHALF)], coli)
                pltpu.async_copy(hs_hbm.at[rowi.at[0]], rows0, gsem)

            @pl.loop(0, NHALF, step=2)
            def _win(j):
                pltpu.make_async_copy(hs_hbm.at[rowi.at[j]], rows0, gsem).wait()
                pltpu.async_copy(hs_hbm.at[rowi.at[j + 1]], rows1, gsem)
                pltpu.sync_copy(rows0, acc.at[coli.at[j]], add=True)
                pltpu.make_async_copy(hs_hbm.at[rowi.at[j + 1]], rows1,
                                      gsem).wait()

                @pl.when(j + 2 < NHALF)
                def _prefetch():
                    pltpu.async_copy(hs_hbm.at[rowi.at[j + 2]], rows0, gsem)

                pltpu.sync_copy(rows1, acc.at[coli.at[j + 1]], add=True)

        plsc.subcore_barrier()
        pltpu.sync_copy(acc.at[pl.ds(s * ZROWS, ZROWS)],
                        out_hbm.at[c, pl.ds(s * ZROWS, ZROWS)])

    return k(hs, ei4, zeros)


def _dis_from(degt_ref):
    deg = jnp.sum(degt_ref[...], axis=1, keepdims=True) + 1.0
    return lax.rsqrt(deg)


def _matmul_t(a, w):
    return lax.dot_general(a, w, (((1,), (1,)), ((), ())),
                           precision=lax.Precision.HIGHEST)


def _bn_lrelu(t, g_ref, be_ref):
    mean = jnp.mean(t, axis=0, keepdims=True)
    var = jnp.mean((t - mean) ** 2, axis=0, keepdims=True)
    t = (t - mean) * lax.rsqrt(var + 1e-5) * g_ref[...] + be_ref[...]
    return jnp.where(t >= 0, t, 0.1 * t)


def _tc_k1a(x, w1, b1):
    def body(x_ref, w_ref, b_ref, o_ref):
        o_ref[...] = _matmul_t(x_ref[...], w_ref[...]) + b_ref[...]

    return pl.pallas_call(
        body, out_shape=jax.ShapeDtypeStruct((N, D), jnp.float32),
    )(x, w1, b1)


def _tc_k1b(h1raw, degt):
    def body(h_ref, degt_ref, o_ref):
        o_ref[...] = h_ref[...] * _dis_from(degt_ref)

    return pl.pallas_call(
        body, out_shape=jax.ShapeDtypeStruct((N, D), jnp.float32),
    )(h1raw, degt)


def _tc_k2(parts, hs1, degt, g1, be1, w2, b2):
    def body(p_ref, hs_ref, degt_ref, g_ref, be_ref, w_ref, b_ref, o_ref):
        dis = _dis_from(degt_ref)
        t = (p_ref[0, :N] + p_ref[1, :N] + hs_ref[...]) * dis
        t = _bn_lrelu(t, g_ref, be_ref)
        o_ref[...] = (_matmul_t(t, w_ref[...]) + b_ref[...]) * dis

    return pl.pallas_call(
        body, out_shape=jax.ShapeDtypeStruct((N, D), jnp.float32),
    )(parts, hs1, degt, g1, be1, w2, b2)


def _tc_k3(parts, hs2, degt, g2, be2):
    def body(p_ref, hs_ref, degt_ref, g_ref, be_ref, o_ref):
        dis = _dis_from(degt_ref)
        t = (p_ref[0, :N] + p_ref[1, :N] + hs_ref[...]) * dis
        o_ref[...] = _bn_lrelu(t, g_ref, be_ref)

    return pl.pallas_call(
        body, out_shape=jax.ShapeDtypeStruct((N, D), jnp.float32),
    )(parts, hs2, degt, g2, be2)


def kernel(x, edge_index, W1, b1, g1, be1, W2, b2, g2, be2):
    ei2 = edge_index.reshape(2, NW, EPW)

    npad = EPW_PAD - EPW
    ar = jnp.arange(npad, dtype=jnp.int32)
    pads = jnp.broadcast_to(
        jnp.stack([(ar * 89) % N, N + ar % (N_PAD - N)])[:, None, :],
        (2, NW, npad))
    ei4 = jnp.concatenate([ei2, pads], axis=2).reshape(2, NW, NWIN, WIN)

    zeros = jnp.zeros((ZROWS, D), jnp.float32)
    b1r, g1r, be1r = b1.reshape(1, D), g1.reshape(1, D), be1.reshape(1, D)
    b2r, g2r, be2r = b2.reshape(1, D), g2.reshape(1, D), be2.reshape(1, D)

    h1raw = _tc_k1a(x, W1, b1r)
    degp = _sc_degree(ei2)
    degt = degp.T

    hs1 = _tc_k1b(h1raw, degt)
    p1 = _sc_aggregate(hs1, ei4, zeros)
    hs2 = _tc_k2(p1, hs1, degt, g1r, be1r, W2, b2r)
    p2 = _sc_aggregate(hs2, ei4, zeros)
    return _tc_k3(p2, hs2, degt, g2r, be2r)


# speedup vs baseline: 1.0672x; 1.0131x over previous
import dataclasses
import functools

import jax
import jax.numpy as jnp
from jax import lax
from jax.experimental import pallas as pl
from jax.experimental.pallas import tpu as pltpu
from jax.experimental.pallas import tpu_sc as plsc

N = 10000
E = 320000
D = 128
NC, NS = 2, 16
NW = NC * NS
EPW = E // NW
WIN = 128
NWIN = 80
NHALF = NWIN // 2
EPW_PAD = NWIN * WIN
N_PAD = 10112
ZROWS = N_PAD // NS

_mesh = plsc.VectorSubcoreMesh(core_axis_name="c", subcore_axis_name="s")

_sc_params = pltpu.CompilerParams()
if "needs_layout_passes" in pltpu.CompilerParams.__dataclass_fields__:
    _sc_params = dataclasses.replace(_sc_params, needs_layout_passes=False)


def _sc_degree(ei2):
    @functools.partial(
        pl.kernel,
        out_type=jax.ShapeDtypeStruct((NW, N), jnp.float32),
        mesh=_mesh,
        compiler_params=_sc_params,
        scratch_types=[
            pltpu.VMEM((EPW,), jnp.int32),
            pltpu.VMEM((N,), jnp.float32),
        ],
    )
    def k(ei_hbm, out_hbm, colv, degv):
        c = lax.axis_index("c")
        s = lax.axis_index("s")
        wid = s * NC + c
        pltpu.sync_copy(ei_hbm.at[1, wid], colv)

        @pl.loop(0, N, step=16)
        def _zero(i):
            degv[pl.ds(i, 16)] = jnp.zeros((16,), jnp.float32)

        ones = jnp.full((16,), 1.0, jnp.float32)

        @pl.loop(0, EPW, step=16)
        def _hist(i):
            plsc.addupdate_scatter(degv, [colv[pl.ds(i, 16)]], ones)

        pltpu.sync_copy(degv, out_hbm.at[wid])

    return k(ei2)


def _sc_aggregate(hs, ei4, zeros):
    @functools.partial(
        pl.kernel,
        out_type=jax.ShapeDtypeStruct((NC, N_PAD, D), jnp.float32),
        mesh=_mesh,
        scratch_types=[
            pltpu.VMEM_SHARED((N_PAD, D), jnp.float32),
            pltpu.VMEM((NHALF, WIN), jnp.int32),
            pltpu.VMEM((NHALF, WIN), jnp.int32),
            pltpu.VMEM((WIN, D), jnp.float32),
            pltpu.VMEM((WIN, D), jnp.float32),
            pltpu.SemaphoreType.DMA,
            pltpu.SemaphoreType.DMA,
        ],
    )
    def k(hs_hbm, ei_hbm, zero_hbm, out_hbm, acc, rowi, coli,
          rows0, rows1, gsem, zsem):
        c = lax.axis_index("c")
        s = lax.axis_index("s")
        wid = s * NC + c
        zcp = pltpu.async_copy(zero_hbm.at[pl.ds(s * ZROWS, ZROWS)],
                               acc.at[pl.ds(s * ZROWS, ZROWS)], zsem)
        pltpu.sync_copy(ei_hbm.at[0, wid, pl.ds(0, NHALF)], rowi)
        pltpu.sync_copy(ei_hbm.at[1, wid, pl.ds(0, NHALF)], coli)
        pltpu.async_copy(hs_hbm.at[rowi.at[0]], rows0, gsem)
        zcp.wait()
        plsc.subcore_barrier()

        for h in range(2):
            if h == 1:
                pltpu.sync_copy(ei_hbm.at[0, wid, pl.ds(NHALF, NHALF)], rowi)
                pltpu.sync_copy(ei_hbm.at[1, wid, pl.ds(NHALF, NHALF)], coli)
                pltpu.async_copy(hs_hbm.at[rowi.at[0]], rows0, gsem)

            @pl.loop(0, NHALF, step=2)
            def _win(j):
                pltpu.make_async_copy(hs_hbm.at[rowi.at[j]], rows0, gsem).wait()
                pltpu.async_copy(hs_hbm.at[rowi.at[j + 1]], rows1, gsem)
                pltpu.sync_copy(rows0, acc.at[coli.at[j]], add=True)
                pltpu.make_async_copy(hs_hbm.at[rowi.at[j + 1]], rows1,
                                      gsem).wait()

                @pl.when(j + 2 < NHALF)
                def _prefetch():
                    pltpu.async_copy(hs_hbm.at[rowi.at[j + 2]], rows0, gsem)

                pltpu.sync_copy(rows1, acc.at[coli.at[j + 1]], add=True)

        plsc.subcore_barrier()
        pltpu.sync_copy(acc.at[pl.ds(s * ZROWS, ZROWS)],
                        out_hbm.at[c, pl.ds(s * ZROWS, ZROWS)])

    return k(hs, ei4, zeros)


def _dis_from(degt_ref):
    deg = jnp.sum(degt_ref[...], axis=1, keepdims=True) + 1.0
    return lax.rsqrt(deg)


def _matmul_t(a, w):
    return lax.dot_general(a, w, (((1,), (1,)), ((), ())),
                           precision=lax.Precision.HIGHEST)


def _bn_lrelu(t, g_ref, be_ref):
    mean = jnp.mean(t, axis=0, keepdims=True)
    var = jnp.mean((t - mean) ** 2, axis=0, keepdims=True)
    t = (t - mean) * lax.rsqrt(var + 1e-5) * g_ref[...] + be_ref[...]
    return jnp.where(t >= 0, t, 0.1 * t)


def _tc_k1a(x, w1, b1):
    def body(x_ref, w_ref, b_ref, o_ref):
        o_ref[...] = _matmul_t(x_ref[...], w_ref[...]) + b_ref[...]

    return pl.pallas_call(
        body, out_shape=jax.ShapeDtypeStruct((N, D), jnp.float32),
    )(x, w1, b1)


def _tc_k1b(h1raw, degt):
    def body(h_ref, degt_ref, o_ref):
        o_ref[...] = h_ref[...] * _dis_from(degt_ref)

    return pl.pallas_call(
        body, out_shape=jax.ShapeDtypeStruct((N, D), jnp.float32),
    )(h1raw, degt)


def _tc_k2(parts, hs1, degt, g1, be1, w2, b2):
    def body(p_ref, hs_ref, degt_ref, g_ref, be_ref, w_ref, b_ref, o_ref):
        dis = _dis_from(degt_ref)
        t = (p_ref[0, :N] + p_ref[1, :N] + hs_ref[...]) * dis
        t = _bn_lrelu(t, g_ref, be_ref)
        o_ref[...] = (_matmul_t(t, w_ref[...]) + b_ref[...]) * dis

    return pl.pallas_call(
        body, out_shape=jax.ShapeDtypeStruct((N, D), jnp.float32),
    )(parts, hs1, degt, g1, be1, w2, b2)


def _tc_k3(parts, hs2, degt, g2, be2):
    def body(p_ref, hs_ref, degt_ref, g_ref, be_ref, o_ref):
        dis = _dis_from(degt_ref)
        t = (p_ref[0, :N] + p_ref[1, :N] + hs_ref[...]) * dis
        o_ref[...] = _bn_lrelu(t, g_ref, be_ref)

    return pl.pallas_call(
        body, out_shape=jax.ShapeDtypeStruct((N, D), jnp.float32),
    )(parts, hs2, degt, g2, be2)


def kernel(x, edge_index, W1, b1, g1, be1, W2, b2, g2, be2):
    ei2 = edge_index.reshape(2, NW, EPW)

    npad = EPW_PAD - EPW
    ar = jnp.arange(npad, dtype=jnp.int32)
    pads = jnp.broadcast_to(
        jnp.stack([(ar * 89) % N, N + ar % (N_PAD - N)])[:, None, :],
        (2, NW, npad))
    ei4 = jnp.concatenate([ei2, pads], axis=2).reshape(2, NW, NWIN, WIN)

    zeros = jnp.zeros((N_PAD, D), jnp.float32)
    b1r, g1r, be1r = b1.reshape(1, D), g1.reshape(1, D), be1.reshape(1, D)
    b2r, g2r, be2r = b2.reshape(1, D), g2.reshape(1, D), be2.reshape(1, D)

    h1raw = _tc_k1a(x, W1, b1r)
    degp = _sc_degree(ei2)
    degt = degp.T

    hs1 = _tc_k1b(h1raw, degt)
    p1 = _sc_aggregate(hs1, ei4, zeros)
    hs2 = _tc_k2(p1, hs1, degt, g1r, be1r, W2, b2r)
    p2 = _sc_aggregate(hs2, ei4, zeros)
    return _tc_k3(p2, hs2, degt, g2r, be2r)
